# baseline (speedup 1.0000x reference)
import functools

import jax
import jax.numpy as jnp
from jax import lax
from jax.experimental import pallas as pl
from jax.experimental.pallas import tpu as pltpu
from jax.experimental.pallas import tpu_sc as plsc

B, S, L, D = 1024, 50, 20, 64
V = 100000
TB = 512
BS = B * S

NC, NS = 2, 16
NW = NC * NS
COLS_W = BS // NW
CCH = 64
NCHUNK = COLS_W // CCH
NAUX = 5
TROWS = CCH * L
AROWS = CCH * NAUX
IB = 128
NG_T = TROWS // IB
AB = 80
NG_A = AROWS // AB

R_TC = 3200
G_TC = BS // R_TC


def _sc_gather_sum(tid_flat, auxf_flat, token_table, aux_table):
  mesh = plsc.VectorSubcoreMesh(core_axis_name="c", subcore_axis_name="s")

  @functools.partial(
      pl.kernel,
      mesh=mesh,
      out_type=jax.ShapeDtypeStruct((BS, D), jnp.float32),
      scratch_types=[
          pltpu.VMEM((NG_T, IB), jnp.int32),
          pltpu.VMEM((NG_A, AB), jnp.int32),
          pltpu.VMEM((TROWS, D // 2), jnp.int32),
          pltpu.VMEM((AROWS, D), jnp.float32),
          pltpu.VMEM((CCH, D), jnp.float32),
          pltpu.SemaphoreType.DMA,
          pltpu.SemaphoreType.DMA,
          pltpu.SemaphoreType.DMA,
          pltpu.SemaphoreType.DMA,
      ],
      compiler_params=pltpu.CompilerParams(use_tc_tiling_on_sc=False,
                                           needs_layout_passes=False),
  )
  def body(tid_hbm, aux_hbm, table_hbm, auxtab_hbm, out_hbm, idx_t, idx_a,
           buf_t, buf_a, out_v, sem0, sem1, sem2, sem3):
    sems = (sem0, sem1, sem2, sem3)
    wid = lax.axis_index("s") * NC + lax.axis_index("c")
    base = wid * COLS_W

    def chunk(ci, carry):
      col0 = base + ci * CCH
      for j in range(NG_T):
        pltpu.sync_copy(tid_hbm.at[pl.ds(col0 * L + j * IB, IB)], idx_t.at[j])
      for j in range(NG_A):
        pltpu.sync_copy(aux_hbm.at[pl.ds(col0 * NAUX + j * AB, AB)],
                        idx_a.at[j])
      descs = []
      for j in range(NG_T):
        descs.append(
            pltpu.async_copy(
                table_hbm.at[idx_t.at[j]], buf_t.at[pl.ds(j * IB, IB)],
                sems[j % 4]))
      for j in range(NG_A):
        descs.append(
            pltpu.async_copy(
                auxtab_hbm.at[idx_a.at[j]], buf_a.at[pl.ds(j * AB, AB)],
                sems[j % 4]))
      for dsc in descs:
        dsc.wait()

      hi_mask = jnp.full((16,), -65536, jnp.int32)

      def col(i, c2):
        tb = i * L
        ab = i * NAUX
        accs = []
        for h in range(2):
          dj = pl.ds(16 * h, 16)
          w = buf_t[tb, dj]
          ev = plsc.bitcast(lax.shift_left(w, 16), jnp.float32)
          od = plsc.bitcast(lax.bitwise_and(w, hi_mask), jnp.float32)
          for l in range(1, L):
            w = buf_t[tb + l, dj]
            ev = ev + plsc.bitcast(lax.shift_left(w, 16), jnp.float32)
            od = od + plsc.bitcast(lax.bitwise_and(w, hi_mask), jnp.float32)
          accs += [ev, od]
        for j in range(4):
          dj = pl.ds(16 * j, 16)
          a = accs[j]
          for q in range(NAUX):
            a = a + buf_a[ab + q, dj]
          out_v[i, dj] = a
        return c2

      lax.fori_loop(0, CCH, col, 0)
      pltpu.sync_copy(out_v, out_hbm.at[pl.ds(col0, CCH)])
      return carry

    lax.fori_loop(0, NCHUNK, chunk, 0)

  return body(tid_flat, auxf_flat, token_table, aux_table)


def _tc_finish(pre, gamma2, beta2, W, b2):
  def body(pre_ref, g_ref, be_ref, w_ref, b_ref, o_ref):
    x = pre_ref[...] * (1.0 / L)
    mu = jnp.mean(x, axis=1, keepdims=True)
    xc = x - mu
    var = jnp.mean(xc * xc, axis=1, keepdims=True)
    nx = xc * lax.rsqrt(var + 1e-5) * g_ref[...] + be_ref[...]
    y = lax.dot_general(nx, w_ref[...], (((1,), (1,)), ((), ())),
                        preferred_element_type=jnp.float32,
                        precision=lax.Precision.HIGHEST)
    o_ref[...] = y + b_ref[...]

  return pl.pallas_call(
      body,
      grid=(G_TC,),
      in_specs=[
          pl.BlockSpec((R_TC, D), lambda i: (i, 0)),
          pl.BlockSpec((1, D), lambda i: (0, 0)),
          pl.BlockSpec((1, D), lambda i: (0, 0)),
          pl.BlockSpec((D, D), lambda i: (0, 0)),
          pl.BlockSpec((1, D), lambda i: (0, 0)),
      ],
      out_specs=pl.BlockSpec((R_TC, D), lambda i: (i, 0)),
      out_shape=jax.ShapeDtypeStruct((BS, D), jnp.float32),
  )(pre, gamma2, beta2, W, b2)


def kernel(token_ids, section, temporality, negated, timestamp_bucket,
           token_table, section_table, temporality_table, negation_table,
           position_table, timestamp_table, ln_gamma, ln_beta, W, b):
  tid_flat = token_ids.astype(jnp.int32).reshape(BS * L)

  tt_i32 = jax.lax.bitcast_convert_type(
      token_table.astype(jnp.bfloat16).reshape(V, D // 2, 2),
      jnp.int32)
  perm = jnp.concatenate([
      jnp.arange(0, 32, 2), jnp.arange(1, 32, 2),
      jnp.arange(32, 64, 2), jnp.arange(33, 64, 2)
  ])

  aux_table = jnp.concatenate([
      section_table, temporality_table, negation_table, timestamp_table,
      position_table
  ], axis=0)[:, perm] * float(L)
  pos_idx = jnp.tile(jnp.arange(S, dtype=jnp.int32), B) + (6 + 3 + 2 + TB)
  auxf_flat = jnp.stack([
      section.astype(jnp.int32).reshape(BS),
      temporality.astype(jnp.int32).reshape(BS) + 6,
      negated.astype(jnp.int32).reshape(BS) + 9,
      timestamp_bucket.astype(jnp.int32).reshape(BS) + 11,
      pos_idx,
  ], axis=1).reshape(BS * NAUX)

  pre = _sc_gather_sum(tid_flat, auxf_flat, tt_i32, aux_table)
  tokens_flat = _tc_finish(pre, ln_gamma[perm].reshape(1, D),
                           ln_beta[perm].reshape(1, D), W[:, perm],
                           b.reshape(1, D))
  tokens = tokens_flat.reshape(B, S, D)
  padding_mask = jnp.zeros((B, S), dtype=bool)
  return tokens, padding_mask


# speedup vs baseline: 3.8985x; 3.8985x over previous
import functools

import jax
import jax.numpy as jnp
from jax import lax
from jax.experimental import pallas as pl
from jax.experimental.pallas import tpu as pltpu
from jax.experimental.pallas import tpu_sc as plsc

B, S, L, D = 1024, 50, 20, 64
V = 100000
TB = 512
BS = B * S

NC, NS = 2, 16
NW = NC * NS
COLS_W = BS // NW
CCH = 80
NCHUNK = COLS_W // CCH
LG = L + 1

R_TC = 3200
G_TC = BS // R_TC


def _sc_gather_sum(idx_t, big_table):
  mesh = plsc.VectorSubcoreMesh(core_axis_name="c", subcore_axis_name="s")

  @functools.partial(
      pl.kernel,
      mesh=mesh,
      out_type=jax.ShapeDtypeStruct((BS, D), jnp.float32),
      scratch_types=[
          pltpu.VMEM((LG, COLS_W), jnp.int32),
          pltpu.VMEM((CCH, D), jnp.float32),
          pltpu.SemaphoreType.DMA,
          pltpu.SemaphoreType.DMA,
      ],
      compiler_params=pltpu.CompilerParams(use_tc_tiling_on_sc=False),
  )
  def body(idx_hbm, table_hbm, out_hbm, idx_v, acc_v, sem, sem2):
    wid = lax.axis_index("s") * NC + lax.axis_index("c")
    base = wid * COLS_W
    for l in range(LG):
      pltpu.sync_copy(idx_hbm.at[l, pl.ds(base, COLS_W)], idx_v.at[l])

    def chunk(ci, carry):
      off = ci * CCH
      pltpu.async_copy(
          table_hbm.at[idx_v.at[0, pl.ds(off, CCH)]], acc_v, sem).wait()
      descs = []
      for l in range(1, LG):
        descs.append(
            pltpu.async_copy(
                table_hbm.at[idx_v.at[l, pl.ds(off, CCH)]], acc_v, sem2,
                add=True))
      for dsc in descs:
        dsc.wait()
      pltpu.sync_copy(acc_v, out_hbm.at[pl.ds(base + off, CCH)])
      return carry

    lax.fori_loop(0, NCHUNK, chunk, 0)

  return body(idx_t, big_table)


def _tc_finish(pre, pos_tiled, gamma2, beta2, W, b2):
  def body(pre_ref, pos_ref, g_ref, be_ref, w_ref, b_ref, o_ref):
    x = pre_ref[...] * (1.0 / L) + pos_ref[...]
    mu = jnp.mean(x, axis=1, keepdims=True)
    xc = x - mu
    var = jnp.mean(xc * xc, axis=1, keepdims=True)
    nx = xc * lax.rsqrt(var + 1e-5) * g_ref[...] + be_ref[...]
    y = lax.dot_general(nx, w_ref[...], (((1,), (1,)), ((), ())),
                        preferred_element_type=jnp.float32,
                        precision=lax.Precision.HIGHEST)
    o_ref[...] = y + b_ref[...]

  return pl.pallas_call(
      body,
      grid=(G_TC,),
      in_specs=[
          pl.BlockSpec((R_TC, D), lambda i: (i, 0)),
          pl.BlockSpec((R_TC, D), lambda i: (0, 0)),
          pl.BlockSpec((1, D), lambda i: (0, 0)),
          pl.BlockSpec((1, D), lambda i: (0, 0)),
          pl.BlockSpec((D, D), lambda i: (0, 0)),
          pl.BlockSpec((1, D), lambda i: (0, 0)),
      ],
      out_specs=pl.BlockSpec((R_TC, D), lambda i: (i, 0)),
      out_shape=jax.ShapeDtypeStruct((BS, D), jnp.float32),
  )(pre, pos_tiled, gamma2, beta2, W, b2)


def kernel(token_ids, section, temporality, negated, timestamp_bucket,
           token_table, section_table, temporality_table, negation_table,
           position_table, timestamp_table, ln_gamma, ln_beta, W, b):
  comb_table = (section_table[:, None, None, None, :]
                + temporality_table[None, :, None, None, :]
                + negation_table[None, None, :, None, :]
                + timestamp_table[None, None, None, :, :]
                ).reshape(6 * 3 * 2 * TB, D) * float(L)
  big_table = jnp.concatenate([token_table, comb_table], axis=0)

  comb_idx = (((section.astype(jnp.int32) * 3 + temporality.astype(jnp.int32))
               * 2 + negated.astype(jnp.int32)) * TB
              + timestamp_bucket.astype(jnp.int32)).reshape(BS) + V

  idx_t = jnp.concatenate([
      token_ids.astype(jnp.int32).reshape(BS, L), comb_idx[:, None]
  ], axis=1).T

  pre = _sc_gather_sum(idx_t, big_table)

  pos_tiled = jnp.tile(position_table, (R_TC // S, 1))
  tokens_flat = _tc_finish(pre, pos_tiled, ln_gamma.reshape(1, D),
                           ln_beta.reshape(1, D), W, b.reshape(1, D))
  tokens = tokens_flat.reshape(B, S, D)
  padding_mask = jnp.zeros((B, S), dtype=bool)
  return tokens, padding_mask


# speedup vs baseline: 4.4469x; 1.1407x over previous
import functools

import jax
import jax.numpy as jnp
from jax import lax
from jax.experimental import pallas as pl
from jax.experimental.pallas import tpu as pltpu
from jax.experimental.pallas import tpu_sc as plsc

B, S, L, D = 1024, 50, 20, 64
V = 100000
TB = 512
BS = B * S

NC, NS = 2, 16
NW = NC * NS
COLS_W = BS // NW
CCH = 80
NCHUNK = COLS_W // CCH
LG = L + 1

R_TC = 3200
G_TC = BS // R_TC


def _sc_gather_sum(idx_t, token_table, comb_table):
  mesh = plsc.VectorSubcoreMesh(core_axis_name="c", subcore_axis_name="s")

  @functools.partial(
      pl.kernel,
      mesh=mesh,
      out_type=jax.ShapeDtypeStruct((BS, D), jnp.float32),
      scratch_types=[
          pltpu.VMEM((LG, COLS_W), jnp.int32),
          pltpu.VMEM((2, CCH, D), jnp.float32),
          pltpu.SemaphoreType.DMA,
          pltpu.SemaphoreType.DMA,
          pltpu.SemaphoreType.DMA,
          pltpu.SemaphoreType.DMA,
          pltpu.SemaphoreType.DMA,
          pltpu.SemaphoreType.DMA,
      ],
      compiler_params=pltpu.CompilerParams(use_tc_tiling_on_sc=False),
  )
  def body(idx_hbm, ttab_hbm, ctab_hbm, out_hbm, idx_v, acc_v, sl0_0, sl0_1,
           sadd_0, sadd_1, sout_0, sout_1):
    s_l0 = (sl0_0, sl0_1)
    s_add = (sadd_0, sadd_1)
    s_out = (sout_0, sout_1)
    wid = lax.axis_index("s") * NC + lax.axis_index("c")
    base = wid * COLS_W
    for l in range(LG):
      pltpu.sync_copy(idx_hbm.at[l, pl.ds(base, COLS_W)], idx_v.at[l])

    def l0_copy(off, p):
      return pltpu.make_async_copy(
          ttab_hbm.at[idx_v.at[0, pl.ds(off, CCH)]], acc_v.at[p], s_l0[p])

    def out_copy(off, p):
      return pltpu.make_async_copy(
          acc_v.at[p], out_hbm.at[pl.ds(base + off, CCH)], s_out[p])

    l0_copy(0, 0).start()

    def step(k2, carry):
      for p in range(2):
        q = 1 - p
        k = k2 * 2 + p
        off = k * CCH
        l0_copy(off, p).wait()
        descs = []
        for l in range(1, L):
          descs.append(
              pltpu.async_copy(
                  ttab_hbm.at[idx_v.at[l, pl.ds(off, CCH)]], acc_v.at[p],
                  s_add[p], add=True))
        descs.append(
            pltpu.async_copy(
                ctab_hbm.at[idx_v.at[L, pl.ds(off, CCH)]], acc_v.at[p],
                s_add[p], add=True))
        @pl.when(k >= 1)
        def _():
          out_copy((k - 1) * CCH, q).wait()

        @pl.when(k + 1 < NCHUNK)
        def _():
          l0_copy((k + 1) * CCH, q).start()

        for dsc in descs:
          dsc.wait()
        out_copy(off, p).start()
      return carry

    lax.fori_loop(0, NCHUNK // 2, step, 0)
    out_copy((NCHUNK - 1) * CCH, 1).wait()

  return body(idx_t, token_table, comb_table)


def _tc_finish(pre, pos_tiled, gamma2, beta2, W, b2):
  def body(pre_ref, pos_ref, g_ref, be_ref, w_ref, b_ref, o_ref):
    x = pre_ref[...] * (1.0 / L) + pos_ref[...]
    mu = jnp.mean(x, axis=1, keepdims=True)
    xc = x - mu
    var = jnp.mean(xc * xc, axis=1, keepdims=True)
    nx = xc * lax.rsqrt(var + 1e-5) * g_ref[...] + be_ref[...]
    y = lax.dot_general(nx, w_ref[...], (((1,), (1,)), ((), ())),
                        preferred_element_type=jnp.float32,
                        precision=lax.Precision.HIGHEST)
    o_ref[...] = y + b_ref[...]

  return pl.pallas_call(
      body,
      grid=(G_TC,),
      in_specs=[
          pl.BlockSpec((R_TC, D), lambda i: (i, 0)),
          pl.BlockSpec((R_TC, D), lambda i: (0, 0)),
          pl.BlockSpec((1, D), lambda i: (0, 0)),
          pl.BlockSpec((1, D), lambda i: (0, 0)),
          pl.BlockSpec((D, D), lambda i: (0, 0)),
          pl.BlockSpec((1, D), lambda i: (0, 0)),
      ],
      out_specs=pl.BlockSpec((R_TC, D), lambda i: (i, 0)),
      out_shape=jax.ShapeDtypeStruct((BS, D), jnp.float32),
  )(pre, pos_tiled, gamma2, beta2, W, b2)


def kernel(token_ids, section, temporality, negated, timestamp_bucket,
           token_table, section_table, temporality_table, negation_table,
           position_table, timestamp_table, ln_gamma, ln_beta, W, b):
  comb_table = (section_table[:, None, None, None, :]
                + temporality_table[None, :, None, None, :]
                + negation_table[None, None, :, None, :]
                + timestamp_table[None, None, None, :, :]
                ).reshape(6 * 3 * 2 * TB, D) * float(L)

  comb_idx = (((section.astype(jnp.int32) * 3 + temporality.astype(jnp.int32))
               * 2 + negated.astype(jnp.int32)) * TB
              + timestamp_bucket.astype(jnp.int32)).reshape(BS)

  idx_t = jnp.concatenate([
      token_ids.astype(jnp.int32).reshape(BS, L), comb_idx[:, None]
  ], axis=1).T

  pre = _sc_gather_sum(idx_t, token_table, comb_table)

  pos_tiled = jnp.tile(position_table, (R_TC // S, 1))
  tokens_flat = _tc_finish(pre, pos_tiled, ln_gamma.reshape(1, D),
                           ln_beta.reshape(1, D), W, b.reshape(1, D))
  tokens = tokens_flat.reshape(B, S, D)
  padding_mask = jnp.zeros((B, S), dtype=bool)
  return tokens, padding_mask


# speedup vs baseline: 4.6065x; 1.0359x over previous
import functools

import jax
import jax.numpy as jnp
from jax import lax
from jax.experimental import pallas as pl
from jax.experimental.pallas import tpu as pltpu
from jax.experimental.pallas import tpu_sc as plsc

B, S, L, D = 1024, 50, 20, 64
V = 100000
TB = 512
BS = B * S

NC, NS = 2, 16
NW = NC * NS
COLS_W = BS // NW
CCH = 80
NCHUNK = COLS_W // CCH
LG = L + 1

R_TC = 3200
G_TC = BS // R_TC


def _sc_gather_sum(idx_t, token_table, comb_table):
  mesh = plsc.VectorSubcoreMesh(core_axis_name="c", subcore_axis_name="s")

  @functools.partial(
      pl.kernel,
      mesh=mesh,
      out_type=jax.ShapeDtypeStruct((BS, D), jnp.float32),
      scratch_types=[
          pltpu.VMEM((LG, COLS_W), jnp.int32),
          pltpu.VMEM((2, CCH, D), jnp.float32),
          pltpu.SemaphoreType.DMA,
          pltpu.SemaphoreType.DMA,
          pltpu.SemaphoreType.DMA,
          pltpu.SemaphoreType.DMA,
          pltpu.SemaphoreType.DMA,
          pltpu.SemaphoreType.DMA,
          pltpu.SemaphoreType.DMA,
      ],
      compiler_params=pltpu.CompilerParams(use_tc_tiling_on_sc=False),
  )
  def body(idx_hbm, ttab_hbm, ctab_hbm, out_hbm, idx_v, acc_v, sl0_0, sl0_1,
           sadd_0, sadd_1, sout_0, sout_1, s_stage):
    s_l0 = (sl0_0, sl0_1)
    s_add = (sadd_0, sadd_1)
    s_out = (sout_0, sout_1)
    wid = lax.axis_index("s") * NC + lax.axis_index("c")
    base = wid * COLS_W

    def l0_copy(off, p):
      return pltpu.make_async_copy(
          ttab_hbm.at[idx_v.at[0, pl.ds(off, CCH)]], acc_v.at[p], s_l0[p])

    def out_copy(off, p):
      return pltpu.make_async_copy(
          acc_v.at[p], out_hbm.at[pl.ds(base + off, CCH)], s_out[p])

    pltpu.async_copy(idx_hbm.at[0, pl.ds(base, COLS_W)], idx_v.at[0],
                     s_stage).wait()
    l0_copy(0, 0).start()
    stage = [
        pltpu.async_copy(idx_hbm.at[l, pl.ds(base, COLS_W)], idx_v.at[l],
                         s_stage) for l in range(1, LG)
    ]
    for dsc in stage:
      dsc.wait()

    def step(k2, carry):
      for p in range(2):
        q = 1 - p
        k = k2 * 2 + p
        off = k * CCH
        l0_copy(off, p).wait()
        descs = []
        for l in range(1, L):
          descs.append(
              pltpu.async_copy(
                  ttab_hbm.at[idx_v.at[l, pl.ds(off, CCH)]], acc_v.at[p],
                  s_add[p], add=True))
        descs.append(
            pltpu.async_copy(
                ctab_hbm.at[idx_v.at[L, pl.ds(off, CCH)]], acc_v.at[p],
                s_add[p], add=True))
        @pl.when(k >= 1)
        def _():
          out_copy((k - 1) * CCH, q).wait()

        @pl.when(k + 1 < NCHUNK)
        def _():
          l0_copy((k + 1) * CCH, q).start()

        for dsc in descs:
          dsc.wait()
        out_copy(off, p).start()
      return carry

    lax.fori_loop(0, NCHUNK // 2, step, 0)
    out_copy((NCHUNK - 1) * CCH, 1).wait()

  return body(idx_t, token_table, comb_table)


def _tc_finish(pre, pos_tiled, gamma2, beta2, W, b2):
  def body(pre_ref, pos_ref, g_ref, be_ref, w_ref, b_ref, o_ref):
    x = pre_ref[...] * (1.0 / L) + pos_ref[...]
    mu = jnp.mean(x, axis=1, keepdims=True)
    xc = x - mu
    var = jnp.mean(xc * xc, axis=1, keepdims=True)
    nx = xc * lax.rsqrt(var + 1e-5) * g_ref[...] + be_ref[...]
    y = lax.dot_general(nx, w_ref[...], (((1,), (1,)), ((), ())),
                        preferred_element_type=jnp.float32,
                        precision=lax.Precision.HIGHEST)
    o_ref[...] = y + b_ref[...]

  return pl.pallas_call(
      body,
      grid=(G_TC,),
      in_specs=[
          pl.BlockSpec((R_TC, D), lambda i: (i, 0)),
          pl.BlockSpec((R_TC, D), lambda i: (0, 0)),
          pl.BlockSpec((1, D), lambda i: (0, 0)),
          pl.BlockSpec((1, D), lambda i: (0, 0)),
          pl.BlockSpec((D, D), lambda i: (0, 0)),
          pl.BlockSpec((1, D), lambda i: (0, 0)),
      ],
      out_specs=pl.BlockSpec((R_TC, D), lambda i: (i, 0)),
      out_shape=jax.ShapeDtypeStruct((BS, D), jnp.float32),
  )(pre, pos_tiled, gamma2, beta2, W, b2)


def kernel(token_ids, section, temporality, negated, timestamp_bucket,
           token_table, section_table, temporality_table, negation_table,
           position_table, timestamp_table, ln_gamma, ln_beta, W, b):
  comb_table = (section_table[:, None, None, None, :]
                + temporality_table[None, :, None, None, :]
                + negation_table[None, None, :, None, :]
                + timestamp_table[None, None, None, :, :]
                ).reshape(6 * 3 * 2 * TB, D) * float(L)

  comb_idx = (((section.astype(jnp.int32) * 3 + temporality.astype(jnp.int32))
               * 2 + negated.astype(jnp.int32)) * TB
              + timestamp_bucket.astype(jnp.int32)).reshape(BS)

  idx_t = jnp.concatenate([
      token_ids.astype(jnp.int32).reshape(BS, L), comb_idx[:, None]
  ], axis=1).T

  pre = _sc_gather_sum(idx_t, token_table, comb_table)

  pos_tiled = jnp.tile(position_table, (R_TC // S, 1))
  tokens_flat = _tc_finish(pre, pos_tiled, ln_gamma.reshape(1, D),
                           ln_beta.reshape(1, D), W, b.reshape(1, D))
  tokens = tokens_flat.reshape(B, S, D)
  padding_mask = jnp.zeros((B, S), dtype=bool)
  return tokens, padding_mask


# speedup vs baseline: 4.8378x; 1.0502x over previous
import functools

import jax
import jax.numpy as jnp
from jax import lax
from jax.experimental import pallas as pl
from jax.experimental.pallas import tpu as pltpu
from jax.experimental.pallas import tpu_sc as plsc

B, S, L, D = 1024, 50, 20, 64
V = 100000
TB = 512
BS = B * S

NC, NS = 2, 16
NW = NC * NS
COLS_W = BS // NW
CCH = 80
NCHUNK = COLS_W // CCH
LG = L + 1

R_TC = 3200
G_TC = BS // R_TC


def _sc_gather_sum(idx_t, token_table, comb_table):
  mesh = plsc.VectorSubcoreMesh(core_axis_name="c", subcore_axis_name="s")

  @functools.partial(
      pl.kernel,
      mesh=mesh,
      out_type=jax.ShapeDtypeStruct((BS, D), jnp.float32),
      scratch_types=[
          pltpu.VMEM((LG, COLS_W), jnp.int32),
          pltpu.VMEM((2, CCH, D), jnp.float32),
          pltpu.SemaphoreType.DMA,
          pltpu.SemaphoreType.DMA,
          pltpu.SemaphoreType.DMA,
          pltpu.SemaphoreType.DMA,
          pltpu.SemaphoreType.DMA,
          pltpu.SemaphoreType.DMA,
          pltpu.SemaphoreType.DMA,
      ],
      compiler_params=pltpu.CompilerParams(use_tc_tiling_on_sc=False),
  )
  def body(idx_hbm, ttab_hbm, ctab_hbm, out_hbm, idx_v, acc_v, sl0_0, sl0_1,
           sadd_0, sadd_1, sout_0, sout_1, s_stage):
    s_l0 = (sl0_0, sl0_1)
    s_add = (sadd_0, sadd_1)
    s_out = (sout_0, sout_1)
    wid = lax.axis_index("s") * NC + lax.axis_index("c")
    base = wid * COLS_W

    def l0_copy(off, p):
      return pltpu.make_async_copy(
          ttab_hbm.at[idx_v.at[0, pl.ds(off, CCH)]], acc_v.at[p], s_l0[p])

    def out_copy(off, p):
      return pltpu.make_async_copy(
          acc_v.at[p], out_hbm.at[pl.ds(base + off, CCH)], s_out[p])

    pltpu.async_copy(idx_hbm.at[0, pl.ds(base, COLS_W)], idx_v.at[0],
                     s_stage).wait()
    l0_copy(0, 0).start()
    stage = [
        pltpu.async_copy(idx_hbm.at[l, pl.ds(base, COLS_W)], idx_v.at[l],
                         s_stage) for l in range(1, LG)
    ]
    for dsc in stage:
      dsc.wait()

    def step(k2, carry):
      for p in range(2):
        q = 1 - p
        k = k2 * 2 + p
        off = k * CCH
        l0_copy(off, p).wait()
        descs = []
        for l in range(1, L):
          descs.append(
              pltpu.async_copy(
                  ttab_hbm.at[idx_v.at[l, pl.ds(off, CCH)]], acc_v.at[p],
                  s_add[p], add=True))
        descs.append(
            pltpu.async_copy(
                ctab_hbm.at[idx_v.at[L, pl.ds(off, CCH)]], acc_v.at[p],
                s_add[p], add=True))
        @pl.when(k >= 1)
        def _():
          out_copy((k - 1) * CCH, q).wait()

        @pl.when(k + 1 < NCHUNK)
        def _():
          l0_copy((k + 1) * CCH, q).start()

        for dsc in descs:
          dsc.wait()
        out_copy(off, p).start()
      return carry

    lax.fori_loop(0, NCHUNK // 2, step, 0)
    out_copy((NCHUNK - 1) * CCH, 1).wait()

  return body(idx_t, token_table, comb_table)


def _tc_finish(pre, pos_tiled, gamma2, beta2, W, b2):
  def body(pre_ref, pos_ref, g_ref, be_ref, w_ref, b_ref, o_ref):
    x = pre_ref[...] * (1.0 / L) + pos_ref[...]
    mu = jnp.mean(x, axis=1, keepdims=True)
    xc = x - mu
    var = jnp.mean(xc * xc, axis=1, keepdims=True)
    nx = xc * lax.rsqrt(var + 1e-5) * g_ref[...] + be_ref[...]
    y = lax.dot_general(nx, w_ref[...], (((1,), (1,)), ((), ())),
                        preferred_element_type=jnp.float32,
                        precision=lax.Precision.HIGHEST)
    o_ref[...] = (y + b_ref[...]).reshape(R_TC // S, S, D)

  return pl.pallas_call(
      body,
      grid=(G_TC,),
      in_specs=[
          pl.BlockSpec((R_TC, D), lambda i: (i, 0)),
          pl.BlockSpec((R_TC, D), lambda i: (0, 0)),
          pl.BlockSpec((1, D), lambda i: (0, 0)),
          pl.BlockSpec((1, D), lambda i: (0, 0)),
          pl.BlockSpec((D, D), lambda i: (0, 0)),
          pl.BlockSpec((1, D), lambda i: (0, 0)),
      ],
      out_specs=pl.BlockSpec((R_TC // S, S, D), lambda i: (i, 0, 0)),
      out_shape=jax.ShapeDtypeStruct((B, S, D), jnp.float32),
  )(pre, pos_tiled, gamma2, beta2, W, b2)


def kernel(token_ids, section, temporality, negated, timestamp_bucket,
           token_table, section_table, temporality_table, negation_table,
           position_table, timestamp_table, ln_gamma, ln_beta, W, b):
  comb_table = (section_table[:, None, None, None, :]
                + temporality_table[None, :, None, None, :]
                + negation_table[None, None, :, None, :]
                + timestamp_table[None, None, None, :, :]
                ).reshape(6 * 3 * 2 * TB, D) * float(L)

  comb_idx = (((section.astype(jnp.int32) * 3 + temporality.astype(jnp.int32))
               * 2 + negated.astype(jnp.int32)) * TB
              + timestamp_bucket.astype(jnp.int32)).reshape(BS)

  idx_t = jnp.concatenate([
      token_ids.astype(jnp.int32).reshape(BS, L), comb_idx[:, None]
  ], axis=1).T

  pre = _sc_gather_sum(idx_t, token_table, comb_table)

  pos_tiled = jnp.tile(position_table, (R_TC // S, 1))
  tokens = _tc_finish(pre, pos_tiled, ln_gamma.reshape(1, D),
                      ln_beta.reshape(1, D), W, b.reshape(1, D))
  padding_mask = jnp.zeros((B, S), dtype=bool)
  return tokens, padding_mask
